# baseline (speedup 1.0000x reference)
import functools

import jax
import jax.numpy as jnp
from jax import lax
from jax.experimental import pallas as pl
from jax.experimental.pallas import tpu as pltpu
from jax.experimental.pallas import tpu_sc as plsc

D = 1024
H = 4096
E = 8
K = 2
B = 2048
A = B * K
EP = 128
T = 128
NT = A // T + E
P = NT * T
HC = 512
NH = H // HC

NC, NS = 2, 16
NW = NC * NS
TPW = B // NW

_NEG = -1e30


def _gating_body(x_ref, gw_ref, gb_ref, loss_ref, slots_ref, gates_ref,
                 etile_ref, act_ref, ranks_ref, ea_ref):
    f32 = jnp.float32
    logits = jnp.dot(x_ref[...], gw_ref[...], preferred_element_type=f32)
    logits = logits + gb_ref[...]
    col = lax.broadcasted_iota(jnp.int32, (B, EP), 1)
    valid = col < E
    lm = jnp.where(valid, logits, _NEG)

    m0 = jnp.max(lm, axis=1, keepdims=True)
    i0 = jnp.min(jnp.where(lm == m0, col, EP), axis=1, keepdims=True)
    lx = jnp.where(col == i0, _NEG, lm)
    m1 = jnp.max(lx, axis=1, keepdims=True)
    i1 = jnp.min(jnp.where(lx == m1, col, EP), axis=1, keepdims=True)

    e1 = jnp.exp(m1 - m0)
    g0 = 1.0 / (1.0 + e1)
    g1 = e1 / (1.0 + e1)

    pfull = jnp.exp(lm - m0)
    probs = pfull / jnp.sum(pfull, axis=1, keepdims=True)
    mean_probs = jnp.sum(probs, axis=0, keepdims=True) * (1.0 / B)
    dif = jnp.where(lax.broadcasted_iota(jnp.int32, (1, EP), 1) < E,
                    mean_probs - (1.0 / E), 0.0)
    loss_ref[...] = jnp.sum(dif * dif).reshape(1, 1)

    r128 = lax.broadcasted_iota(jnp.int32, (128, 128), 0)
    c128 = lax.broadcasted_iota(jnp.int32, (128, 128), 1)
    tril = (r128 > c128).astype(f32)
    triu = (r128 < c128).astype(f32)

    ea_ref[pl.ds(0, B), :] = i0
    ea_ref[pl.ds(B, B), :] = i1

    def chunk(c, run):
        base = c * 128
        ea = ea_ref[pl.ds(base, 128), :]
        onehot = (ea == lax.broadcasted_iota(jnp.int32, (128, EP), 1)).astype(f32)
        pre = jnp.dot(tril, onehot, preferred_element_type=f32) + run
        rank = jnp.sum(pre * onehot, axis=1, keepdims=True)
        ranks_ref[pl.ds(c * 128, 128), :] = rank
        return run + jnp.sum(onehot, axis=0, keepdims=True)

    counts = lax.fori_loop(0, A // 128, chunk, jnp.zeros((1, EP), f32))

    ecol = lax.broadcasted_iota(jnp.int32, (1, EP), 1)
    padded = jnp.where(ecol < E,
                       jnp.floor((counts + (T - 1)) * (1.0 / T)) * T, 0.0)
    offs = jnp.dot(padded, triu, preferred_element_type=f32)

    def slot_chunk(c, _):
        base = c * 128
        ea = ea_ref[pl.ds(base, 128), :]
        onehot = (ea == lax.broadcasted_iota(jnp.int32, (128, EP), 1)).astype(f32)
        off_j = jnp.dot(onehot, offs.reshape(EP, 1), preferred_element_type=f32)
        rank = ranks_ref[pl.ds(base, 128), :]
        slots_ref[pl.ds(base, 128), :] = (off_j + rank).astype(jnp.int32)
        return 0

    lax.fori_loop(0, A // 128, slot_chunk, 0)

    gates_ref[pl.ds(0, B), :] = jnp.broadcast_to(g0, (B, 16))
    gates_ref[pl.ds(B, B), :] = jnp.broadcast_to(g1, (B, 16))

    tstart = lax.broadcasted_iota(jnp.int32, (NT, EP), 0).astype(f32) * T
    offb = jnp.broadcast_to(offs, (NT, EP))
    pb = jnp.broadcast_to(padded, (NT, EP))
    emask = ((tstart >= offb) & (tstart < offb + pb) &
             (lax.broadcasted_iota(jnp.int32, (NT, EP), 1) < E))
    emf = emask.astype(f32)
    etile_ref[...] = jnp.sum(
        emf * lax.broadcasted_iota(jnp.int32, (NT, EP), 1).astype(f32),
        axis=1, keepdims=True
    ).astype(jnp.int32)
    act_ref[...] = jnp.sum(emf, axis=1, keepdims=True).astype(jnp.int32)


def _gating(x, gwp, gbp):
    return pl.pallas_call(
        _gating_body,
        out_shape=(
            jax.ShapeDtypeStruct((1, 1), jnp.float32),
            jax.ShapeDtypeStruct((A, 1), jnp.int32),
            jax.ShapeDtypeStruct((A, 16), jnp.float32),
            jax.ShapeDtypeStruct((NT, 1), jnp.int32),
            jax.ShapeDtypeStruct((NT, 1), jnp.int32),
        ),
        scratch_shapes=[pltpu.VMEM((A, 1), jnp.float32),
                        pltpu.VMEM((A, 1), jnp.int32)],
    )(x, gwp, gbp)


def _ffn_body(et_ref, act_ref, xs_ref, w1_ref, b1_ref, w2_ref, b2_ref, out_ref):
    j = pl.program_id(0)
    i = pl.program_id(1)

    @pl.when(act_ref[i] == 1)
    def _():
        row = pl.ds(i * T, T)
        xb = xs_ref[row, :].astype(jnp.bfloat16)
        w1 = w1_ref[0].astype(jnp.bfloat16)
        h = jnp.dot(xb, w1, preferred_element_type=jnp.float32)
        h = jnp.maximum(h + b1_ref[0], 0.0)
        w2 = w2_ref[0].astype(jnp.bfloat16)
        acc = jnp.dot(h.astype(jnp.bfloat16), w2,
                      preferred_element_type=jnp.float32)

        @pl.when(j == 0)
        def _():
            out_ref[row, :] = acc + b2_ref[0]

        @pl.when(j != 0)
        def _():
            out_ref[row, :] += acc


def _ffn(e_of_tile, active, xs, W1, b1, W2, b2):
    grid_spec = pltpu.PrefetchScalarGridSpec(
        num_scalar_prefetch=2,
        grid=(NH, NT),
        in_specs=[
            pl.BlockSpec((P, D), lambda j, i, et, at: (0, 0)),
            pl.BlockSpec((1, D, HC), lambda j, i, et, at: (et[i], 0, j)),
            pl.BlockSpec((1, 1, HC), lambda j, i, et, at: (et[i], 0, j)),
            pl.BlockSpec((1, HC, D), lambda j, i, et, at: (et[i], j, 0)),
            pl.BlockSpec((1, 1, D), lambda j, i, et, at: (et[i], 0, 0)),
        ],
        out_specs=pl.BlockSpec((P, D), lambda j, i, et, at: (0, 0)),
    )
    return pl.pallas_call(
        _ffn_body,
        grid_spec=grid_spec,
        out_shape=jax.ShapeDtypeStruct((P, D), jnp.float32),
        compiler_params=pltpu.CompilerParams(
            dimension_semantics=("arbitrary", "arbitrary"),
        ),
    )(e_of_tile, active, xs, W1, b1.reshape(E, 1, H), W2, b2.reshape(E, 1, D))


def _dispatch_sc(x, slots):
    mesh = plsc.VectorSubcoreMesh(core_axis_name="c", subcore_axis_name="s")

    @functools.partial(
        pl.kernel,
        mesh=mesh,
        out_type=jax.ShapeDtypeStruct((P, D), jnp.float32),
        scratch_types=[
            pltpu.VMEM((TPW,), jnp.int32),
            pltpu.VMEM((TPW,), jnp.int32),
            pltpu.VMEM((TPW, D), jnp.float32),
            pltpu.SemaphoreType.DMA,
            pltpu.SemaphoreType.DMA,
        ],
    )
    def dispatch(x_hbm, slots_hbm, xs_hbm, idx0_v, idx1_v, rows_v, s0, s1):
        wid = lax.axis_index("s") * NC + lax.axis_index("c")
        base = wid * TPW
        pltpu.sync_copy(slots_hbm.at[pl.ds(base, TPW)], idx0_v)
        pltpu.sync_copy(slots_hbm.at[pl.ds(B + base, TPW)], idx1_v)
        pltpu.sync_copy(x_hbm.at[pl.ds(base, TPW)], rows_v)
        c0 = pltpu.make_async_copy(rows_v, xs_hbm.at[idx0_v], s0)
        c1 = pltpu.make_async_copy(rows_v, xs_hbm.at[idx1_v], s1)
        c0.start()
        c1.start()
        c0.wait()
        c1.wait()

    return dispatch(x, slots)


CH = 16


def _combine_sc(ys, slots, gates):
    mesh = plsc.VectorSubcoreMesh(core_axis_name="c", subcore_axis_name="s")

    @functools.partial(
        pl.kernel,
        mesh=mesh,
        out_type=jax.ShapeDtypeStruct((B, D), jnp.float32),
        scratch_types=[
            pltpu.VMEM((CH,), jnp.int32),
            pltpu.VMEM((CH,), jnp.int32),
            pltpu.VMEM((CH, 16), jnp.float32),
            pltpu.VMEM((CH, 16), jnp.float32),
            pltpu.VMEM((CH, D), jnp.float32),
            pltpu.VMEM((CH, D), jnp.float32),
            pltpu.VMEM((CH, D), jnp.float32),
            pltpu.SemaphoreType.DMA,
            pltpu.SemaphoreType.DMA,
        ],
    )
    def combine(ys_hbm, slots_hbm, gates_hbm, out_hbm,
                idx0_v, idx1_v, g0_v, g1_v, r0_v, r1_v, o_v, s0, s1):
        wid = lax.axis_index("s") * NC + lax.axis_index("c")
        base = wid * TPW

        def chunk(c, _):
            tb = base + c * CH
            pltpu.sync_copy(slots_hbm.at[pl.ds(tb, CH)], idx0_v)
            pltpu.sync_copy(slots_hbm.at[pl.ds(B + tb, CH)], idx1_v)
            pltpu.sync_copy(gates_hbm.at[pl.ds(tb, CH)], g0_v)
            pltpu.sync_copy(gates_hbm.at[pl.ds(B + tb, CH)], g1_v)
            c0 = pltpu.make_async_copy(ys_hbm.at[idx0_v], r0_v, s0)
            c1 = pltpu.make_async_copy(ys_hbm.at[idx1_v], r1_v, s1)
            c0.start()
            c1.start()
            c0.wait()
            c1.wait()

            def token(t, _):
                g0 = g0_v[t, pl.ds(0, 16)]
                g1 = g1_v[t, pl.ds(0, 16)]
                for v in range(D // 16):
                    sl = pl.ds(v * 16, 16)
                    o_v[t, sl] = g0 * r0_v[t, sl] + g1 * r1_v[t, sl]
                return 0

            lax.fori_loop(0, CH, token, 0)
            pltpu.sync_copy(o_v, out_hbm.at[pl.ds(tb, CH)])
            return 0

        lax.fori_loop(0, TPW // CH, chunk, 0)

    return combine(ys, slots, gates)


def kernel(x, gate_W, gate_b, W1, b1, W2, b2):
    gwp = jnp.pad(gate_W, ((0, 0), (0, EP - E)))
    gbp = jnp.pad(gate_b, (0, EP - E)).reshape(1, EP)
    loss, slots2, gates2, etile, act = _gating(x, gwp, gbp)
    slots = slots2.reshape(A)
    xs = _dispatch_sc(x, slots)
    ys = _ffn(etile.reshape(NT), act.reshape(NT), xs, W1, b1, W2, b2)
    out = _combine_sc(ys, slots, gates2)
    return out, loss.reshape(())


# speedup vs baseline: 1.0747x; 1.0747x over previous
import functools

import jax
import jax.numpy as jnp
from jax import lax
from jax.experimental import pallas as pl
from jax.experimental.pallas import tpu as pltpu
from jax.experimental.pallas import tpu_sc as plsc

D = 1024
H = 4096
E = 8
K = 2
B = 2048
A = B * K
EP = 128
T = 256
NT = A // T + E
P = NT * T
HC = 512
NH = H // HC

NC, NS = 2, 16
NW = NC * NS
TPW = B // NW

_NEG = -1e30


def _gating_body(x_ref, gw_ref, gb_ref, loss_ref, slots_ref, gates_ref,
                 etile_ref, act_ref, ranks_ref, ea_ref):
    f32 = jnp.float32
    logits = jnp.dot(x_ref[...], gw_ref[...], preferred_element_type=f32)
    logits = logits + gb_ref[...]
    col = lax.broadcasted_iota(jnp.int32, (B, EP), 1)
    valid = col < E
    lm = jnp.where(valid, logits, _NEG)

    m0 = jnp.max(lm, axis=1, keepdims=True)
    i0 = jnp.min(jnp.where(lm == m0, col, EP), axis=1, keepdims=True)
    lx = jnp.where(col == i0, _NEG, lm)
    m1 = jnp.max(lx, axis=1, keepdims=True)
    i1 = jnp.min(jnp.where(lx == m1, col, EP), axis=1, keepdims=True)

    e1 = jnp.exp(m1 - m0)
    g0 = 1.0 / (1.0 + e1)
    g1 = e1 / (1.0 + e1)

    pfull = jnp.exp(lm - m0)
    probs = pfull / jnp.sum(pfull, axis=1, keepdims=True)
    mean_probs = jnp.sum(probs, axis=0, keepdims=True) * (1.0 / B)
    dif = jnp.where(lax.broadcasted_iota(jnp.int32, (1, EP), 1) < E,
                    mean_probs - (1.0 / E), 0.0)
    loss_ref[...] = jnp.sum(dif * dif).reshape(1, 1)

    r128 = lax.broadcasted_iota(jnp.int32, (128, 128), 0)
    c128 = lax.broadcasted_iota(jnp.int32, (128, 128), 1)
    tril = (r128 > c128).astype(f32)
    triu = (r128 < c128).astype(f32)

    ea_ref[pl.ds(0, B), :] = i0
    ea_ref[pl.ds(B, B), :] = i1

    def chunk(c, run):
        base = c * 128
        ea = ea_ref[pl.ds(base, 128), :]
        onehot = (ea == lax.broadcasted_iota(jnp.int32, (128, EP), 1)).astype(f32)
        pre = jnp.dot(tril, onehot, preferred_element_type=f32) + run
        rank = jnp.sum(pre * onehot, axis=1, keepdims=True)
        ranks_ref[pl.ds(c * 128, 128), :] = rank
        return run + jnp.sum(onehot, axis=0, keepdims=True)

    counts = lax.fori_loop(0, A // 128, chunk, jnp.zeros((1, EP), f32))

    ecol = lax.broadcasted_iota(jnp.int32, (1, EP), 1)
    padded = jnp.where(ecol < E,
                       jnp.floor((counts + (T - 1)) * (1.0 / T)) * T, 0.0)
    offs = jnp.dot(padded, triu, preferred_element_type=f32)

    def slot_chunk(c, _):
        base = c * 128
        ea = ea_ref[pl.ds(base, 128), :]
        onehot = (ea == lax.broadcasted_iota(jnp.int32, (128, EP), 1)).astype(f32)
        off_j = jnp.dot(onehot, offs.reshape(EP, 1), preferred_element_type=f32)
        rank = ranks_ref[pl.ds(base, 128), :]
        slots_ref[pl.ds(base, 128), :] = (off_j + rank).astype(jnp.int32)
        return 0

    lax.fori_loop(0, A // 128, slot_chunk, 0)

    gates_ref[pl.ds(0, B), :] = jnp.broadcast_to(g0, (B, 16))
    gates_ref[pl.ds(B, B), :] = jnp.broadcast_to(g1, (B, 16))

    tstart = lax.broadcasted_iota(jnp.int32, (NT, EP), 0).astype(f32) * T
    offb = jnp.broadcast_to(offs, (NT, EP))
    pb = jnp.broadcast_to(padded, (NT, EP))
    emask = ((tstart >= offb) & (tstart < offb + pb) &
             (lax.broadcasted_iota(jnp.int32, (NT, EP), 1) < E))
    emf = emask.astype(f32)
    etile_ref[...] = jnp.sum(
        emf * lax.broadcasted_iota(jnp.int32, (NT, EP), 1).astype(f32),
        axis=1, keepdims=True
    ).astype(jnp.int32)
    act_ref[...] = jnp.sum(emf, axis=1, keepdims=True).astype(jnp.int32)


def _gating(x, gwp, gbp):
    return pl.pallas_call(
        _gating_body,
        out_shape=(
            jax.ShapeDtypeStruct((1, 1), jnp.float32),
            jax.ShapeDtypeStruct((A, 1), jnp.int32),
            jax.ShapeDtypeStruct((A, 16), jnp.float32),
            jax.ShapeDtypeStruct((NT, 1), jnp.int32),
            jax.ShapeDtypeStruct((NT, 1), jnp.int32),
        ),
        scratch_shapes=[pltpu.VMEM((A, 1), jnp.float32),
                        pltpu.VMEM((A, 1), jnp.int32)],
    )(x, gwp, gbp)


def _ffn_body(et_ref, act_ref, xs_ref, w1_ref, b1_ref, w2_ref, b2_ref, out_ref):
    i = pl.program_id(0)
    j = pl.program_id(1)

    @pl.when(act_ref[i] == 1)
    def _():
        xb = xs_ref[...].astype(jnp.bfloat16)
        w1 = w1_ref[0].astype(jnp.bfloat16)
        h = jnp.dot(xb, w1, preferred_element_type=jnp.float32)
        h = jnp.maximum(h + b1_ref[0], 0.0)
        w2 = w2_ref[0].astype(jnp.bfloat16)
        acc = jnp.dot(h.astype(jnp.bfloat16), w2,
                      preferred_element_type=jnp.float32)

        @pl.when(j == 0)
        def _():
            out_ref[...] = acc + b2_ref[0]

        @pl.when(j != 0)
        def _():
            out_ref[...] += acc


def _ffn(e_of_tile, active, xs, W1, b1, W2, b2):
    grid_spec = pltpu.PrefetchScalarGridSpec(
        num_scalar_prefetch=2,
        grid=(NT, NH),
        in_specs=[
            pl.BlockSpec((T, D), lambda i, j, et, at: (i, 0)),
            pl.BlockSpec((1, D, HC), lambda i, j, et, at: (et[i], 0, j)),
            pl.BlockSpec((1, 1, HC), lambda i, j, et, at: (et[i], 0, j)),
            pl.BlockSpec((1, HC, D), lambda i, j, et, at: (et[i], j, 0)),
            pl.BlockSpec((1, 1, D), lambda i, j, et, at: (et[i], 0, 0)),
        ],
        out_specs=pl.BlockSpec((T, D), lambda i, j, et, at: (i, 0)),
    )
    return pl.pallas_call(
        _ffn_body,
        grid_spec=grid_spec,
        out_shape=jax.ShapeDtypeStruct((P, D), jnp.float32),
    )(e_of_tile, active, xs, W1, b1.reshape(E, 1, H), W2, b2.reshape(E, 1, D))


def _dispatch_sc(x, slots):
    mesh = plsc.VectorSubcoreMesh(core_axis_name="c", subcore_axis_name="s")

    @functools.partial(
        pl.kernel,
        mesh=mesh,
        out_type=jax.ShapeDtypeStruct((P, D), jnp.float32),
        scratch_types=[
            pltpu.VMEM((TPW,), jnp.int32),
            pltpu.VMEM((TPW,), jnp.int32),
            pltpu.VMEM((TPW, D), jnp.float32),
            pltpu.SemaphoreType.DMA,
            pltpu.SemaphoreType.DMA,
        ],
    )
    def dispatch(x_hbm, slots_hbm, xs_hbm, idx0_v, idx1_v, rows_v, s0, s1):
        wid = lax.axis_index("s") * NC + lax.axis_index("c")
        base = wid * TPW
        pltpu.sync_copy(slots_hbm.at[pl.ds(base, TPW)], idx0_v)
        pltpu.sync_copy(slots_hbm.at[pl.ds(B + base, TPW)], idx1_v)
        pltpu.sync_copy(x_hbm.at[pl.ds(base, TPW)], rows_v)
        c0 = pltpu.make_async_copy(rows_v, xs_hbm.at[idx0_v], s0)
        c1 = pltpu.make_async_copy(rows_v, xs_hbm.at[idx1_v], s1)
        c0.start()
        c1.start()
        c0.wait()
        c1.wait()

    return dispatch(x, slots)


CH = 16


def _combine_sc(ys, slots, gates):
    mesh = plsc.VectorSubcoreMesh(core_axis_name="c", subcore_axis_name="s")

    @functools.partial(
        pl.kernel,
        mesh=mesh,
        out_type=jax.ShapeDtypeStruct((B, D), jnp.float32),
        scratch_types=[
            pltpu.VMEM((CH,), jnp.int32),
            pltpu.VMEM((CH,), jnp.int32),
            pltpu.VMEM((CH, 16), jnp.float32),
            pltpu.VMEM((CH, 16), jnp.float32),
            pltpu.VMEM((CH, D), jnp.float32),
            pltpu.VMEM((CH, D), jnp.float32),
            pltpu.VMEM((CH, D), jnp.float32),
            pltpu.SemaphoreType.DMA,
            pltpu.SemaphoreType.DMA,
        ],
    )
    def combine(ys_hbm, slots_hbm, gates_hbm, out_hbm,
                idx0_v, idx1_v, g0_v, g1_v, r0_v, r1_v, o_v, s0, s1):
        wid = lax.axis_index("s") * NC + lax.axis_index("c")
        base = wid * TPW

        def chunk(c, _):
            tb = base + c * CH
            pltpu.sync_copy(slots_hbm.at[pl.ds(tb, CH)], idx0_v)
            pltpu.sync_copy(slots_hbm.at[pl.ds(B + tb, CH)], idx1_v)
            pltpu.sync_copy(gates_hbm.at[pl.ds(tb, CH)], g0_v)
            pltpu.sync_copy(gates_hbm.at[pl.ds(B + tb, CH)], g1_v)
            c0 = pltpu.make_async_copy(ys_hbm.at[idx0_v], r0_v, s0)
            c1 = pltpu.make_async_copy(ys_hbm.at[idx1_v], r1_v, s1)
            c0.start()
            c1.start()
            c0.wait()
            c1.wait()

            def token(t, _):
                g0 = g0_v[t, pl.ds(0, 16)]
                g1 = g1_v[t, pl.ds(0, 16)]
                for v in range(D // 16):
                    sl = pl.ds(v * 16, 16)
                    o_v[t, sl] = g0 * r0_v[t, sl] + g1 * r1_v[t, sl]
                return 0

            lax.fori_loop(0, CH, token, 0)
            pltpu.sync_copy(o_v, out_hbm.at[pl.ds(tb, CH)])
            return 0

        lax.fori_loop(0, TPW // CH, chunk, 0)

    return combine(ys, slots, gates)


def kernel(x, gate_W, gate_b, W1, b1, W2, b2):
    gwp = jnp.pad(gate_W, ((0, 0), (0, EP - E)))
    gbp = jnp.pad(gate_b, (0, EP - E)).reshape(1, EP)
    loss, slots2, gates2, etile, act = _gating(x, gwp, gbp)
    slots = slots2.reshape(A)
    xs = _dispatch_sc(x, slots)
    ys = _ffn(etile.reshape(NT), act.reshape(NT), xs, W1, b1, W2, b2)
    out = _combine_sc(ys, slots, gates2)
    return out, loss.reshape(())


# speedup vs baseline: 1.3411x; 1.2479x over previous
import functools

import jax
import jax.numpy as jnp
from jax import lax
from jax.experimental import pallas as pl
from jax.experimental.pallas import tpu as pltpu
from jax.experimental.pallas import tpu_sc as plsc

D = 1024
H = 4096
E = 8
K = 2
B = 2048
A = B * K
EP = 128
T = 512
NT = A // T + E
P = NT * T
HC = 512
NH = H // HC

NC, NS = 2, 16
NW = NC * NS
TPW = B // NW

_NEG = -1e30


def _gating_body(x_ref, gw_ref, gb_ref, loss_ref, slots_ref, gates_ref,
                 etile_ref, act_ref, ranks_ref, ea_ref):
    f32 = jnp.float32
    logits = jnp.dot(x_ref[...], gw_ref[...], preferred_element_type=f32)
    logits = logits + gb_ref[...]
    col = lax.broadcasted_iota(jnp.int32, (B, EP), 1)
    valid = col < E
    lm = jnp.where(valid, logits, _NEG)

    m0 = jnp.max(lm, axis=1, keepdims=True)
    i0 = jnp.min(jnp.where(lm == m0, col, EP), axis=1, keepdims=True)
    lx = jnp.where(col == i0, _NEG, lm)
    m1 = jnp.max(lx, axis=1, keepdims=True)
    i1 = jnp.min(jnp.where(lx == m1, col, EP), axis=1, keepdims=True)

    e1 = jnp.exp(m1 - m0)
    g0 = 1.0 / (1.0 + e1)
    g1 = e1 / (1.0 + e1)

    pfull = jnp.exp(lm - m0)
    probs = pfull / jnp.sum(pfull, axis=1, keepdims=True)
    mean_probs = jnp.sum(probs, axis=0, keepdims=True) * (1.0 / B)
    dif = jnp.where(lax.broadcasted_iota(jnp.int32, (1, EP), 1) < E,
                    mean_probs - (1.0 / E), 0.0)
    loss_ref[...] = jnp.sum(dif * dif).reshape(1, 1)

    r128 = lax.broadcasted_iota(jnp.int32, (128, 128), 0)
    c128 = lax.broadcasted_iota(jnp.int32, (128, 128), 1)
    tril = (r128 > c128).astype(f32)
    triu = (r128 < c128).astype(f32)

    ea_ref[pl.ds(0, B), :] = i0
    ea_ref[pl.ds(B, B), :] = i1

    def chunk(c, run):
        base = c * 128
        ea = ea_ref[pl.ds(base, 128), :]
        onehot = (ea == lax.broadcasted_iota(jnp.int32, (128, EP), 1)).astype(f32)
        pre = jnp.dot(tril, onehot, preferred_element_type=f32) + run
        rank = jnp.sum(pre * onehot, axis=1, keepdims=True)
        ranks_ref[pl.ds(c * 128, 128), :] = rank
        return run + jnp.sum(onehot, axis=0, keepdims=True)

    counts = lax.fori_loop(0, A // 128, chunk, jnp.zeros((1, EP), f32))

    ecol = lax.broadcasted_iota(jnp.int32, (1, EP), 1)
    padded = jnp.where(ecol < E,
                       jnp.floor((counts + (T - 1)) * (1.0 / T)) * T, 0.0)
    offs = jnp.dot(padded, triu, preferred_element_type=f32)

    def slot_chunk(c, _):
        base = c * 128
        ea = ea_ref[pl.ds(base, 128), :]
        onehot = (ea == lax.broadcasted_iota(jnp.int32, (128, EP), 1)).astype(f32)
        off_j = jnp.dot(onehot, offs.reshape(EP, 1), preferred_element_type=f32)
        rank = ranks_ref[pl.ds(base, 128), :]
        slots_ref[pl.ds(base, 128), :] = (off_j + rank).astype(jnp.int32)
        return 0

    lax.fori_loop(0, A // 128, slot_chunk, 0)

    gates_ref[pl.ds(0, B), :] = jnp.broadcast_to(g0, (B, 16))
    gates_ref[pl.ds(B, B), :] = jnp.broadcast_to(g1, (B, 16))

    tstart = lax.broadcasted_iota(jnp.int32, (NT, EP), 0).astype(f32) * T
    offb = jnp.broadcast_to(offs, (NT, EP))
    pb = jnp.broadcast_to(padded, (NT, EP))
    emask = ((tstart >= offb) & (tstart < offb + pb) &
             (lax.broadcasted_iota(jnp.int32, (NT, EP), 1) < E))
    emf = emask.astype(f32)
    etile_ref[...] = jnp.sum(
        emf * lax.broadcasted_iota(jnp.int32, (NT, EP), 1).astype(f32),
        axis=1, keepdims=True
    ).astype(jnp.int32)
    act_ref[...] = jnp.sum(emf, axis=1, keepdims=True).astype(jnp.int32)


def _gating(x, gwp, gbp):
    return pl.pallas_call(
        _gating_body,
        out_shape=(
            jax.ShapeDtypeStruct((1, 1), jnp.float32),
            jax.ShapeDtypeStruct((A, 1), jnp.int32),
            jax.ShapeDtypeStruct((A, 16), jnp.float32),
            jax.ShapeDtypeStruct((NT, 1), jnp.int32),
            jax.ShapeDtypeStruct((NT, 1), jnp.int32),
        ),
        scratch_shapes=[pltpu.VMEM((A, 1), jnp.float32),
                        pltpu.VMEM((A, 1), jnp.int32)],
    )(x, gwp, gbp)


def _ffn_body(et_ref, act_ref, xs_ref, w1_ref, b1_ref, w2_ref, b2_ref, out_ref):
    i = pl.program_id(0)
    j = pl.program_id(1)

    @pl.when(act_ref[i] == 1)
    def _():
        xb = xs_ref[...].astype(jnp.bfloat16)
        w1 = w1_ref[0].astype(jnp.bfloat16)
        h = jnp.dot(xb, w1, preferred_element_type=jnp.float32)
        h = jnp.maximum(h + b1_ref[0], 0.0)
        w2 = w2_ref[0].astype(jnp.bfloat16)
        acc = jnp.dot(h.astype(jnp.bfloat16), w2,
                      preferred_element_type=jnp.float32)

        @pl.when(j == 0)
        def _():
            out_ref[...] = acc + b2_ref[0]

        @pl.when(j != 0)
        def _():
            out_ref[...] += acc


def _ffn(e_of_tile, active, xs, W1, b1, W2, b2):
    grid_spec = pltpu.PrefetchScalarGridSpec(
        num_scalar_prefetch=2,
        grid=(NT, NH),
        in_specs=[
            pl.BlockSpec((T, D), lambda i, j, et, at: (i, 0)),
            pl.BlockSpec((1, D, HC), lambda i, j, et, at: (et[i], 0, j)),
            pl.BlockSpec((1, 1, HC), lambda i, j, et, at: (et[i], 0, j)),
            pl.BlockSpec((1, HC, D), lambda i, j, et, at: (et[i], j, 0)),
            pl.BlockSpec((1, 1, D), lambda i, j, et, at: (et[i], 0, 0)),
        ],
        out_specs=pl.BlockSpec((T, D), lambda i, j, et, at: (i, 0)),
    )
    return pl.pallas_call(
        _ffn_body,
        grid_spec=grid_spec,
        out_shape=jax.ShapeDtypeStruct((P, D), jnp.float32),
    )(e_of_tile, active, xs, W1, b1.reshape(E, 1, H), W2, b2.reshape(E, 1, D))


def _dispatch_sc(x, slots):
    mesh = plsc.VectorSubcoreMesh(core_axis_name="c", subcore_axis_name="s")

    @functools.partial(
        pl.kernel,
        mesh=mesh,
        out_type=jax.ShapeDtypeStruct((P, D), jnp.float32),
        scratch_types=[
            pltpu.VMEM((TPW,), jnp.int32),
            pltpu.VMEM((TPW,), jnp.int32),
            pltpu.VMEM((TPW, D), jnp.float32),
            pltpu.SemaphoreType.DMA,
            pltpu.SemaphoreType.DMA,
        ],
    )
    def dispatch(x_hbm, slots_hbm, xs_hbm, idx0_v, idx1_v, rows_v, s0, s1):
        wid = lax.axis_index("s") * NC + lax.axis_index("c")
        base = wid * TPW
        pltpu.sync_copy(slots_hbm.at[pl.ds(base, TPW)], idx0_v)
        pltpu.sync_copy(slots_hbm.at[pl.ds(B + base, TPW)], idx1_v)
        pltpu.sync_copy(x_hbm.at[pl.ds(base, TPW)], rows_v)
        c0 = pltpu.make_async_copy(rows_v, xs_hbm.at[idx0_v], s0)
        c1 = pltpu.make_async_copy(rows_v, xs_hbm.at[idx1_v], s1)
        c0.start()
        c1.start()
        c0.wait()
        c1.wait()

    return dispatch(x, slots)


CH = 16


def _combine_sc(ys, slots, gates):
    mesh = plsc.VectorSubcoreMesh(core_axis_name="c", subcore_axis_name="s")

    @functools.partial(
        pl.kernel,
        mesh=mesh,
        out_type=jax.ShapeDtypeStruct((B, D), jnp.float32),
        scratch_types=[
            pltpu.VMEM((CH,), jnp.int32),
            pltpu.VMEM((CH,), jnp.int32),
            pltpu.VMEM((CH, 16), jnp.float32),
            pltpu.VMEM((CH, 16), jnp.float32),
            pltpu.VMEM((CH, D), jnp.float32),
            pltpu.VMEM((CH, D), jnp.float32),
            pltpu.VMEM((CH, D), jnp.float32),
            pltpu.SemaphoreType.DMA,
            pltpu.SemaphoreType.DMA,
        ],
    )
    def combine(ys_hbm, slots_hbm, gates_hbm, out_hbm,
                idx0_v, idx1_v, g0_v, g1_v, r0_v, r1_v, o_v, s0, s1):
        wid = lax.axis_index("s") * NC + lax.axis_index("c")
        base = wid * TPW

        def chunk(c, _):
            tb = base + c * CH
            pltpu.sync_copy(slots_hbm.at[pl.ds(tb, CH)], idx0_v)
            pltpu.sync_copy(slots_hbm.at[pl.ds(B + tb, CH)], idx1_v)
            pltpu.sync_copy(gates_hbm.at[pl.ds(tb, CH)], g0_v)
            pltpu.sync_copy(gates_hbm.at[pl.ds(B + tb, CH)], g1_v)
            c0 = pltpu.make_async_copy(ys_hbm.at[idx0_v], r0_v, s0)
            c1 = pltpu.make_async_copy(ys_hbm.at[idx1_v], r1_v, s1)
            c0.start()
            c1.start()
            c0.wait()
            c1.wait()

            def token(t, _):
                g0 = g0_v[t, pl.ds(0, 16)]
                g1 = g1_v[t, pl.ds(0, 16)]
                for v in range(D // 16):
                    sl = pl.ds(v * 16, 16)
                    o_v[t, sl] = g0 * r0_v[t, sl] + g1 * r1_v[t, sl]
                return 0

            lax.fori_loop(0, CH, token, 0)
            pltpu.sync_copy(o_v, out_hbm.at[pl.ds(tb, CH)])
            return 0

        lax.fori_loop(0, TPW // CH, chunk, 0)

    return combine(ys, slots, gates)


def kernel(x, gate_W, gate_b, W1, b1, W2, b2):
    gwp = jnp.pad(gate_W, ((0, 0), (0, EP - E)))
    gbp = jnp.pad(gate_b, (0, EP - E)).reshape(1, EP)
    loss, slots2, gates2, etile, act = _gating(x, gwp, gbp)
    slots = slots2.reshape(A)
    xs = _dispatch_sc(x, slots)
    ys = _ffn(etile.reshape(NT), act.reshape(NT), xs, W1, b1, W2, b2)
    out = _combine_sc(ys, slots, gates2)
    return out, loss.reshape(())
